# baseline (speedup 1.0000x reference)
import functools

import jax
import jax.numpy as jnp
from jax import lax
from jax.experimental import pallas as pl
from jax.experimental.pallas import tpu as pltpu
from jax.experimental.pallas import tpu_sc as plsc

_B, _H, _C = 32, 32768, 16
_K = 8
_NC, _NS = 2, 16
_L = 16
_CHUNK = 2048
_NCHUNK = _H // _CHUNK
_GROUPS = _CHUNK // (8 * _L)

_SORT8 = (
    (0, 1), (2, 3), (4, 5), (6, 7),
    (0, 2), (1, 3), (4, 6), (5, 7),
    (1, 2), (5, 6),
    (0, 4), (1, 5), (2, 6), (3, 7),
    (2, 4), (3, 5),
    (1, 2), (3, 4), (5, 6),
)
_MERGE8 = (
    (0, 4), (1, 5), (2, 6), (3, 7),
    (0, 2), (1, 3), (4, 6), (5, 7),
    (0, 1), (2, 3), (4, 5), (6, 7),
)


def _sort8(v):
    v = list(v)
    for i, j in _SORT8:
        hi = jnp.maximum(v[i], v[j])
        lo = jnp.minimum(v[i], v[j])
        v[i], v[j] = hi, lo
    return v


def _resort_bitonic(w):
    w = list(w)
    for i, j in _MERGE8:
        hi = jnp.maximum(w[i], w[j])
        lo = jnp.minimum(w[i], w[j])
        w[i], w[j] = hi, lo
    return w


def _merge_top8(r, g):
    return _resort_bitonic([jnp.maximum(r[i], g[7 - i]) for i in range(8)])


@functools.partial(
    pl.kernel,
    out_type=jax.ShapeDtypeStruct((_B, _K, _C), jnp.float32),
    mesh=plsc.VectorSubcoreMesh(core_axis_name="c", subcore_axis_name="s"),
    compiler_params=pltpu.CompilerParams(needs_layout_passes=False),
    scratch_types=[
        pltpu.VMEM((_C, _CHUNK), jnp.float32),
        pltpu.VMEM((_C, _CHUNK), jnp.float32),
        pltpu.VMEM((_C, _K * _L), jnp.float32),
        pltpu.VMEM((_K, _C), jnp.float32),
        pltpu.SemaphoreType.DMA,
        pltpu.SemaphoreType.DMA,
    ],
)
def _kmax_sc(x_hbm, out_hbm, buf0, buf1, acc, out_v, sem0, sem1):
    wid = lax.axis_index("s") * _NC + lax.axis_index("c")
    bufs = (buf0, buf1)
    sems = (sem0, sem1)

    neg_inf = jnp.full((_L,), -jnp.inf, jnp.float32)

    def init_body(c, _):
        for i in range(_K):
            acc[c, pl.ds(i * _L, _L)] = neg_inf
        return 0
    lax.fori_loop(0, _C, init_body, 0)

    copies = {}
    copies[0] = pltpu.async_copy(
        x_hbm.at[wid, :, pl.ds(0, _CHUNK)], buf0, sem0)

    for ci in range(_NCHUNK):
        buf = bufs[ci % 2]
        copies.pop(ci).wait()
        if ci + 1 < _NCHUNK:
            copies[ci + 1] = pltpu.async_copy(
                x_hbm.at[wid, :, pl.ds((ci + 1) * _CHUNK, _CHUNK)],
                bufs[(ci + 1) % 2], sems[(ci + 1) % 2])

        def chan_body(c, _, buf=buf):
            r = tuple(acc[c, pl.ds(i * _L, _L)] for i in range(_K))
            tau = jnp.broadcast_to(jnp.max(r[_K - 1]), (_L,))

            def grp_body(g, carry):
                rs, tau = carry[:_K], carry[_K]
                base = g * (8 * _L)
                v = [buf[c, pl.ds(base + k * _L, _L)] for k in range(8)]
                m01 = jnp.maximum(v[0], v[1])
                m23 = jnp.maximum(v[2], v[3])
                m45 = jnp.maximum(v[4], v[5])
                m67 = jnp.maximum(v[6], v[7])
                gmax = jnp.maximum(jnp.maximum(m01, m23),
                                   jnp.maximum(m45, m67))
                q = jnp.max(jnp.where(gmax > tau, 1, 0))

                def trig():
                    nr = _merge_top8(list(rs), _sort8(v))
                    ntau = jnp.broadcast_to(jnp.max(nr[_K - 1]), (_L,))
                    return tuple(nr) + (ntau,)

                def skip():
                    return carry

                return lax.cond(q > 0, trig, skip)

            rr = lax.fori_loop(0, _GROUPS, grp_body, r + (tau,))
            for i in range(_K):
                acc[c, pl.ds(i * _L, _L)] = rr[i]
            return 0

        lax.fori_loop(0, _C, chan_body, 0)

    lane = lax.iota(jnp.int32, _L)

    def tail_body(c, _):
        r = [acc[c, pl.ds(i * _L, _L)] for i in range(_K)]
        for d in (8, 4, 2, 1):
            perm = lane ^ d
            p = [r[7 - i][perm] for i in range(_K)]
            r = _resort_bitonic([jnp.maximum(r[i], p[i]) for i in range(_K)])
        cmask = lane == c
        for k in range(_K):
            row = out_v[k, :]
            out_v[k, :] = jnp.where(cmask, r[k], row)
        return 0

    lax.fori_loop(0, _C, tail_body, 0)
    pltpu.sync_copy(out_v, out_hbm.at[wid])


def kernel(inputs):
    xt = jnp.transpose(inputs.reshape(_B, _H, _C), (0, 2, 1))
    out = _kmax_sc(xt)
    return out.reshape(_B, _K, 1, _C)


# speedup vs baseline: 2.1766x; 2.1766x over previous
import functools

import jax
import jax.numpy as jnp
from jax import lax
from jax.experimental import pallas as pl
from jax.experimental.pallas import tpu as pltpu
from jax.experimental.pallas import tpu_sc as plsc

_B, _H, _C = 32, 32768, 16
_K = 8
_NC, _NS = 2, 16
_L = 16
_CHUNK = 2048
_NCHUNK = _H // _CHUNK
_GROUPS = _CHUNK // (8 * _L)

_SORT8 = (
    (0, 1), (2, 3), (4, 5), (6, 7),
    (0, 2), (1, 3), (4, 6), (5, 7),
    (1, 2), (5, 6),
    (0, 4), (1, 5), (2, 6), (3, 7),
    (2, 4), (3, 5),
    (1, 2), (3, 4), (5, 6),
)
_MERGE8 = (
    (0, 4), (1, 5), (2, 6), (3, 7),
    (0, 2), (1, 3), (4, 6), (5, 7),
    (0, 1), (2, 3), (4, 5), (6, 7),
)


def _sort8(v):
    v = list(v)
    for i, j in _SORT8:
        hi = jnp.maximum(v[i], v[j])
        lo = jnp.minimum(v[i], v[j])
        v[i], v[j] = hi, lo
    return v


def _resort_bitonic(w):
    w = list(w)
    for i, j in _MERGE8:
        hi = jnp.maximum(w[i], w[j])
        lo = jnp.minimum(w[i], w[j])
        w[i], w[j] = hi, lo
    return w


def _merge_top8(r, g):
    return _resort_bitonic([jnp.maximum(r[i], g[7 - i]) for i in range(8)])


@functools.partial(
    pl.kernel,
    out_type=jax.ShapeDtypeStruct((_B, _K, _C), jnp.float32),
    mesh=plsc.VectorSubcoreMesh(core_axis_name="c", subcore_axis_name="s"),
    scratch_types=[
        pltpu.VMEM((_C, _CHUNK), jnp.float32),
        pltpu.VMEM((_C, _CHUNK), jnp.float32),
        pltpu.VMEM((_C, _K * _L), jnp.float32),
        pltpu.VMEM((_K, _C), jnp.float32),
        pltpu.SemaphoreType.DMA,
        pltpu.SemaphoreType.DMA,
    ],
)
def _kmax_sc(x_hbm, out_hbm, buf0, buf1, acc, out_v, sem0, sem1):
    wid = lax.axis_index("s") * _NC + lax.axis_index("c")
    bufs = (buf0, buf1)
    sems = (sem0, sem1)

    neg_inf = jnp.full((_L,), -jnp.inf, jnp.float32)

    def init_body(c, _):
        for i in range(_K):
            acc[c, pl.ds(i * _L, _L)] = neg_inf
        return 0
    lax.fori_loop(0, _C, init_body, 0)

    copies = {}
    copies[0] = pltpu.async_copy(
        x_hbm.at[wid, :, pl.ds(0, _CHUNK)], buf0, sem0)

    for ci in range(_NCHUNK):
        buf = bufs[ci % 2]
        copies.pop(ci).wait()
        if ci + 1 < _NCHUNK:
            copies[ci + 1] = pltpu.async_copy(
                x_hbm.at[wid, :, pl.ds((ci + 1) * _CHUNK, _CHUNK)],
                bufs[(ci + 1) % 2], sems[(ci + 1) % 2])

        def chan_body(c, _, buf=buf):
            ra = tuple(acc[c, pl.ds(i * _L, _L)] for i in range(_K))
            rb = tuple(acc[c + _C // 2, pl.ds(i * _L, _L)] for i in range(_K))

            def grp_body(g, rs):
                base = g * (8 * _L)
                va = [buf[c, pl.ds(base + k * _L, _L)] for k in range(8)]
                vb = [buf[c + _C // 2, pl.ds(base + k * _L, _L)]
                      for k in range(8)]
                va = _sort8(va)
                vb = _sort8(vb)
                na = _merge_top8(list(rs[:_K]), va)
                nb = _merge_top8(list(rs[_K:]), vb)
                return tuple(na) + tuple(nb)

            rr = lax.fori_loop(0, _GROUPS, grp_body, ra + rb, unroll=2)
            for i in range(_K):
                acc[c, pl.ds(i * _L, _L)] = rr[i]
                acc[c + _C // 2, pl.ds(i * _L, _L)] = rr[_K + i]
            return 0

        lax.fori_loop(0, _C // 2, chan_body, 0)

    lane = lax.iota(jnp.int32, _L)

    def tail_body(c, _):
        r = [acc[c, pl.ds(i * _L, _L)] for i in range(_K)]
        for d in (8, 4, 2, 1):
            perm = lane ^ d
            p = [r[7 - i][perm] for i in range(_K)]
            r = _resort_bitonic([jnp.maximum(r[i], p[i]) for i in range(_K)])
        cmask = lane == c
        for k in range(_K):
            row = out_v[k, :]
            out_v[k, :] = jnp.where(cmask, r[k], row)
        return 0

    lax.fori_loop(0, _C, tail_body, 0)
    pltpu.sync_copy(out_v, out_hbm.at[wid])


def kernel(inputs):
    xt = jnp.transpose(inputs.reshape(_B, _H, _C), (0, 2, 1))
    out = _kmax_sc(xt)
    return out.reshape(_B, _K, 1, _C)


# speedup vs baseline: 2.2695x; 1.0427x over previous
import functools

import jax
import jax.numpy as jnp
from jax import lax
from jax.experimental import pallas as pl
from jax.experimental.pallas import tpu as pltpu
from jax.experimental.pallas import tpu_sc as plsc

_B, _H, _C = 32, 32768, 16
_K = 8
_NC, _NS = 2, 16
_L = 16
_CHUNK = 2048
_NCHUNK = _H // _CHUNK
_GROUPS = _CHUNK // (8 * _L)

_SORT8 = (
    (0, 1), (2, 3), (4, 5), (6, 7),
    (0, 2), (1, 3), (4, 6), (5, 7),
    (1, 2), (5, 6),
    (0, 4), (1, 5), (2, 6), (3, 7),
    (2, 4), (3, 5),
    (1, 2), (3, 4), (5, 6),
)
_MERGE8 = (
    (0, 4), (1, 5), (2, 6), (3, 7),
    (0, 2), (1, 3), (4, 6), (5, 7),
    (0, 1), (2, 3), (4, 5), (6, 7),
)


def _sort8(v):
    v = list(v)
    for i, j in _SORT8:
        hi = jnp.maximum(v[i], v[j])
        lo = jnp.minimum(v[i], v[j])
        v[i], v[j] = hi, lo
    return v


def _resort_bitonic(w):
    w = list(w)
    for i, j in _MERGE8:
        hi = jnp.maximum(w[i], w[j])
        lo = jnp.minimum(w[i], w[j])
        w[i], w[j] = hi, lo
    return w


def _merge_top8(r, g):
    return _resort_bitonic([jnp.maximum(r[i], g[7 - i]) for i in range(8)])


@functools.partial(
    pl.kernel,
    out_type=jax.ShapeDtypeStruct((_B, _K, _C), jnp.float32),
    mesh=plsc.VectorSubcoreMesh(core_axis_name="c", subcore_axis_name="s"),
    scratch_types=[
        pltpu.VMEM((_C, _CHUNK), jnp.float32),
        pltpu.VMEM((_C, _CHUNK), jnp.float32),
        pltpu.VMEM((_C, _K * _L), jnp.float32),
        pltpu.VMEM((_K, _C), jnp.float32),
        pltpu.SemaphoreType.DMA,
        pltpu.SemaphoreType.DMA,
    ],
)
def _kmax_sc(x_hbm, out_hbm, buf0, buf1, acc, out_v, sem0, sem1):
    wid = lax.axis_index("s") * _NC + lax.axis_index("c")
    bufs = (buf0, buf1)
    sems = (sem0, sem1)

    neg_inf = jnp.full((_L,), -jnp.inf, jnp.float32)

    def init_body(c, _):
        for i in range(_K):
            acc[c, pl.ds(i * _L, _L)] = neg_inf
        return 0
    lax.fori_loop(0, _C, init_body, 0)

    copies = {}
    copies[0] = pltpu.async_copy(
        x_hbm.at[wid, :, pl.ds(0, _CHUNK)], buf0, sem0)

    for ci in range(_NCHUNK):
        buf = bufs[ci % 2]
        copies.pop(ci).wait()
        if ci + 1 < _NCHUNK:
            copies[ci + 1] = pltpu.async_copy(
                x_hbm.at[wid, :, pl.ds((ci + 1) * _CHUNK, _CHUNK)],
                bufs[(ci + 1) % 2], sems[(ci + 1) % 2])

        def chan_body(c, _, buf=buf):
            ra = tuple(acc[c, pl.ds(i * _L, _L)] for i in range(_K))
            rb = tuple(acc[c + _C // 2, pl.ds(i * _L, _L)] for i in range(_K))

            def grp_body(g, rs):
                base = g * (8 * _L)
                va = [buf[c, pl.ds(base + k * _L, _L)] for k in range(8)]
                vb = [buf[c + _C // 2, pl.ds(base + k * _L, _L)]
                      for k in range(8)]
                va = _sort8(va)
                vb = _sort8(vb)
                na = _merge_top8(list(rs[:_K]), va)
                nb = _merge_top8(list(rs[_K:]), vb)
                return tuple(na) + tuple(nb)

            rr = lax.fori_loop(0, _GROUPS, grp_body, ra + rb)
            for i in range(_K):
                acc[c, pl.ds(i * _L, _L)] = rr[i]
                acc[c + _C // 2, pl.ds(i * _L, _L)] = rr[_K + i]
            return 0

        lax.fori_loop(0, _C // 2, chan_body, 0)

    lane = lax.iota(jnp.int32, _L)

    def tail_body(c, _):
        r = [acc[c, pl.ds(i * _L, _L)] for i in range(_K)]
        for d in (8, 4, 2, 1):
            perm = lane ^ d
            p = [r[7 - i][perm] for i in range(_K)]
            r = _resort_bitonic([jnp.maximum(r[i], p[i]) for i in range(_K)])
        cmask = lane == c
        for k in range(_K):
            row = out_v[k, :]
            out_v[k, :] = jnp.where(cmask, r[k], row)
        return 0

    lax.fori_loop(0, _C, tail_body, 0)
    pltpu.sync_copy(out_v, out_hbm.at[wid])


def kernel(inputs):
    xt = jnp.transpose(inputs.reshape(_B, _H, _C), (0, 2, 1))
    out = _kmax_sc(xt)
    return out.reshape(_B, _K, 1, _C)
